# Initial kernel scaffold; baseline (speedup 1.0000x reference)
import jax
import jax.numpy as jnp
from jax.experimental import pallas as pl


def kernel(x, edge_index, W1, b1, W2, b2):
    raise NotImplementedError("write your pallas kernel here")


# speedup vs baseline: 8.3332x; 8.3332x over previous
import functools

import jax
import jax.numpy as jnp
from jax import lax
from jax.experimental import pallas as pl
from jax.experimental.pallas import tpu as pltpu
from jax.experimental.pallas import tpu_sc as plsc

_N = 10000
_E = 320000
_NW = 32
_RPT = 640
_NPAD = 10240
_CHUNK = 128
_CH = 80
_EPAD = _NW * _CH * _CHUNK

_mesh = plsc.VectorSubcoreMesh(core_axis_name="c", subcore_axis_name="s")
_sc_params = pltpu.CompilerParams(use_tc_tiling_on_sc=False)


def _deg_body(cols_hbm, zeros_hbm, out_hbm, colv, onesv, acc):
    cid = lax.axis_index("c")
    sid = lax.axis_index("s")
    wid = sid * 2 + cid
    sl = pl.ds(sid * _RPT, _RPT)
    pltpu.sync_copy(zeros_hbm.at[sl], acc.at[sl])
    pltpu.sync_copy(cols_hbm.at[wid], colv)
    for k in range(_CHUNK // 16):
        onesv[pl.ds(k * 16, 16)] = jnp.full((16,), 1.0, jnp.float32)
    plsc.subcore_barrier()

    def body(j, carry):
        pltpu.sync_copy(onesv, acc.at[colv.at[j]], add=True)
        return carry

    lax.fori_loop(0, _CH, body, 0)
    plsc.subcore_barrier()
    pltpu.sync_copy(acc.at[sl], out_hbm.at[cid, sl])


_deg_kernel = functools.partial(
    pl.kernel,
    out_type=jax.ShapeDtypeStruct((2, _NPAD), jnp.float32),
    mesh=_mesh,
    compiler_params=_sc_params,
    scratch_types=[
        pltpu.VMEM((_CH, _CHUNK), jnp.int32),
        pltpu.VMEM((_CHUNK,), jnp.float32),
        pltpu.VMEM_SHARED((_NPAD,), jnp.float32),
    ],
)(_deg_body)


def _make_agg(F):
    def body(y_hbm, rows_hbm, cols_hbm, zeros_hbm, out_hbm,
             rowv, colv, bufa, bufb, sema, semb, acc):
        cid = lax.axis_index("c")
        sid = lax.axis_index("s")
        wid = sid * 2 + cid
        sl = pl.ds(sid * _RPT, _RPT)
        pltpu.sync_copy(zeros_hbm.at[sl], acc.at[sl])
        pltpu.sync_copy(rows_hbm.at[wid], rowv)
        pltpu.sync_copy(cols_hbm.at[wid], colv)
        plsc.subcore_barrier()

        pltpu.async_copy(y_hbm.at[rowv.at[0]], bufa, sema)

        def body2(i, carry):
            j = 2 * i
            pltpu.make_async_copy(y_hbm.at[rowv.at[j]], bufa, sema).wait()
            pltpu.async_copy(y_hbm.at[rowv.at[j + 1]], bufb, semb)
            pltpu.sync_copy(bufa, acc.at[colv.at[j]], add=True)
            pltpu.make_async_copy(y_hbm.at[rowv.at[j + 1]], bufb, semb).wait()
            pltpu.async_copy(y_hbm.at[rowv.at[j + 2]], bufa, sema)
            pltpu.sync_copy(bufb, acc.at[colv.at[j + 1]], add=True)
            return carry

        lax.fori_loop(0, _CH // 2, body2, 0)
        pltpu.make_async_copy(y_hbm.at[rowv.at[_CH]], bufa, sema).wait()
        plsc.subcore_barrier()
        pltpu.sync_copy(acc.at[sl], out_hbm.at[cid, sl])

    return functools.partial(
        pl.kernel,
        out_type=jax.ShapeDtypeStruct((2, _NPAD, F), jnp.float32),
        mesh=_mesh,
        compiler_params=_sc_params,
        scratch_types=[
            pltpu.VMEM((_CH + 1, _CHUNK), jnp.int32),
            pltpu.VMEM((_CH, _CHUNK), jnp.int32),
            pltpu.VMEM((_CHUNK, F), jnp.float32),
            pltpu.VMEM((_CHUNK, F), jnp.float32),
            pltpu.SemaphoreType.DMA,
            pltpu.SemaphoreType.DMA,
            pltpu.VMEM_SHARED((_NPAD, F), jnp.float32),
        ],
    )(body)


_agg64 = _make_agg(64)

_BLK = 2000
_G = _N // _BLK


def _mm1_body(x_ref, w_ref, d0_ref, d1_ref, o_ref):
    dis = lax.rsqrt(d0_ref[...] + d1_ref[...] + 1.0)
    o_ref[...] = dis * jnp.dot(x_ref[...], w_ref[...],
                               preferred_element_type=jnp.float32)


_mm1 = pl.pallas_call(
    _mm1_body,
    grid=(_G,),
    in_specs=[
        pl.BlockSpec((_BLK, 128), lambda i: (i, 0)),
        pl.BlockSpec((128, 64), lambda i: (0, 0)),
        pl.BlockSpec((_BLK, 1), lambda i: (i, 0)),
        pl.BlockSpec((_BLK, 1), lambda i: (i, 0)),
    ],
    out_specs=pl.BlockSpec((_BLK, 64), lambda i: (i, 0)),
    out_shape=jax.ShapeDtypeStruct((_N, 64), jnp.float32),
)


def _mm2_body(p0_ref, p1_ref, y1_ref, d0_ref, d1_ref, b1_ref, w2_ref,
              oa_ref, ob_ref):
    dis = lax.rsqrt(d0_ref[...] + d1_ref[...] + 1.0)
    h = jnp.maximum(dis * (p0_ref[...] + p1_ref[...] + y1_ref[...])
                    + b1_ref[...], 0.0)
    y2 = dis * jnp.dot(h, w2_ref[...], preferred_element_type=jnp.float32)
    oa_ref[...] = y2[:, :64]
    ob_ref[...] = y2[:, 64:]


_mm2 = pl.pallas_call(
    _mm2_body,
    grid=(_G,),
    in_specs=[
        pl.BlockSpec((_BLK, 64), lambda i: (i, 0)),
        pl.BlockSpec((_BLK, 64), lambda i: (i, 0)),
        pl.BlockSpec((_BLK, 64), lambda i: (i, 0)),
        pl.BlockSpec((_BLK, 1), lambda i: (i, 0)),
        pl.BlockSpec((_BLK, 1), lambda i: (i, 0)),
        pl.BlockSpec((1, 64), lambda i: (0, 0)),
        pl.BlockSpec((64, 128), lambda i: (0, 0)),
    ],
    out_specs=[
        pl.BlockSpec((_BLK, 64), lambda i: (i, 0)),
        pl.BlockSpec((_BLK, 64), lambda i: (i, 0)),
    ],
    out_shape=[
        jax.ShapeDtypeStruct((_N, 64), jnp.float32),
        jax.ShapeDtypeStruct((_N, 64), jnp.float32),
    ],
)


def _fin_body(qa0_ref, qa1_ref, qb0_ref, qb1_ref, ya_ref, yb_ref,
              d0_ref, d1_ref, b2_ref, o_ref):
    dis = lax.rsqrt(d0_ref[...] + d1_ref[...] + 1.0)
    oa = dis * (qa0_ref[...] + qa1_ref[...] + ya_ref[...])
    ob = dis * (qb0_ref[...] + qb1_ref[...] + yb_ref[...])
    o_ref[...] = jnp.concatenate([oa, ob], axis=1) + b2_ref[...]


_fin = pl.pallas_call(
    _fin_body,
    grid=(_G,),
    in_specs=[
        pl.BlockSpec((_BLK, 64), lambda i: (i, 0)),
        pl.BlockSpec((_BLK, 64), lambda i: (i, 0)),
        pl.BlockSpec((_BLK, 64), lambda i: (i, 0)),
        pl.BlockSpec((_BLK, 64), lambda i: (i, 0)),
        pl.BlockSpec((_BLK, 64), lambda i: (i, 0)),
        pl.BlockSpec((_BLK, 64), lambda i: (i, 0)),
        pl.BlockSpec((_BLK, 1), lambda i: (i, 0)),
        pl.BlockSpec((_BLK, 1), lambda i: (i, 0)),
        pl.BlockSpec((1, 128), lambda i: (0, 0)),
    ],
    out_specs=pl.BlockSpec((_BLK, 128), lambda i: (i, 0)),
    out_shape=jax.ShapeDtypeStruct((_N, 128), jnp.float32),
)


def kernel(x, edge_index, W1, b1, W2, b2):
    row = edge_index[0].astype(jnp.int32)
    col = edge_index[1].astype(jnp.int32)
    pad = _EPAD - _E
    rowp = jnp.concatenate([row, jnp.zeros((pad,), jnp.int32)])
    colp = jnp.concatenate([col, jnp.full((pad,), _N, jnp.int32)])
    rowp = rowp.reshape(_NW, _CH, _CHUNK)
    colp = colp.reshape(_NW, _CH, _CHUNK)
    rows_g = jnp.concatenate(
        [rowp, jnp.zeros((_NW, 1, _CHUNK), jnp.int32)], axis=1)
    zeros1 = jnp.zeros((_NPAD,), jnp.float32)
    zeros64 = jnp.zeros((_NPAD, 64), jnp.float32)

    deg = _deg_kernel(colp, zeros1)
    d0 = deg[0, :_N, None]
    d1 = deg[1, :_N, None]

    y1 = _mm1(x, W1, d0, d1)
    p = _agg64(y1, rows_g, colp, zeros64)
    ya, yb = _mm2(p[0, :_N], p[1, :_N], y1, d0, d1, b1[None, :], W2)
    qa = _agg64(ya, rows_g, colp, zeros64)
    qb = _agg64(yb, rows_g, colp, zeros64)
    out = _fin(qa[0, :_N], qa[1, :_N], qb[0, :_N], qb[1, :_N],
               ya, yb, d0, d1, b2[None, :])
    return out


# speedup vs baseline: 14.8781x; 1.7854x over previous
import functools

import jax
import jax.numpy as jnp
from jax import lax
from jax.experimental import pallas as pl
from jax.experimental.pallas import tpu as pltpu
from jax.experimental.pallas import tpu_sc as plsc

_N = 10000
_E = 320000
_NW = 32
_RPT = 640
_NPAD = 10240
_CHUNK = 128
_CH = 80
_EPAD = _NW * _CH * _CHUNK

_mesh = plsc.VectorSubcoreMesh(core_axis_name="c", subcore_axis_name="s")
_sc_params = pltpu.CompilerParams(use_tc_tiling_on_sc=False)


def _deg_body(cols_hbm, zeros_hbm, out_hbm, colv, onesv, acc):
    cid = lax.axis_index("c")
    sid = lax.axis_index("s")
    wid = sid * 2 + cid
    sl = pl.ds(sid * _RPT, _RPT)
    pltpu.sync_copy(zeros_hbm.at[sl], acc.at[sl])
    pltpu.sync_copy(cols_hbm.at[wid], colv)
    for k in range(_CHUNK // 16):
        onesv[pl.ds(k * 16, 16)] = jnp.full((16,), 1.0, jnp.float32)
    plsc.subcore_barrier()

    def body(j, carry):
        pltpu.sync_copy(onesv, acc.at[colv.at[j]], add=True)
        return carry

    lax.fori_loop(0, _CH, body, 0)
    plsc.subcore_barrier()
    pltpu.sync_copy(acc.at[sl], out_hbm.at[cid, sl])


_deg_kernel = functools.partial(
    pl.kernel,
    out_type=jax.ShapeDtypeStruct((2, _NPAD), jnp.float32),
    mesh=_mesh,
    compiler_params=_sc_params,
    scratch_types=[
        pltpu.VMEM((_CH, _CHUNK), jnp.int32),
        pltpu.VMEM((_CHUNK,), jnp.float32),
        pltpu.VMEM_SHARED((_NPAD,), jnp.float32),
    ],
)(_deg_body)


def _make_agg(F, dtype):
    def body(y_hbm, rows_hbm, cols_hbm, zeros_hbm, out_hbm,
             rowv, colv, bufa, bufb, sema, semb, acc):
        cid = lax.axis_index("c")
        sid = lax.axis_index("s")
        wid = sid * 2 + cid
        sl = pl.ds(sid * _RPT, _RPT)
        pltpu.sync_copy(zeros_hbm.at[sl], acc.at[sl])
        pltpu.sync_copy(rows_hbm.at[wid], rowv)
        pltpu.sync_copy(cols_hbm.at[wid], colv)
        plsc.subcore_barrier()

        pltpu.async_copy(y_hbm.at[rowv.at[0]], bufa, sema)

        def body2(i, carry):
            j = 2 * i
            pltpu.make_async_copy(y_hbm.at[rowv.at[j]], bufa, sema).wait()
            pltpu.async_copy(y_hbm.at[rowv.at[j + 1]], bufb, semb)
            pltpu.sync_copy(bufa, acc.at[colv.at[j]], add=True)
            pltpu.make_async_copy(y_hbm.at[rowv.at[j + 1]], bufb, semb).wait()
            pltpu.async_copy(y_hbm.at[rowv.at[j + 2]], bufa, sema)
            pltpu.sync_copy(bufb, acc.at[colv.at[j + 1]], add=True)
            return carry

        lax.fori_loop(0, _CH // 2, body2, 0)
        pltpu.make_async_copy(y_hbm.at[rowv.at[_CH]], bufa, sema).wait()
        plsc.subcore_barrier()
        pltpu.sync_copy(acc.at[sl], out_hbm.at[cid, sl])

    return functools.partial(
        pl.kernel,
        out_type=jax.ShapeDtypeStruct((2, _NPAD, F), dtype),
        mesh=_mesh,
        compiler_params=_sc_params,
        scratch_types=[
            pltpu.VMEM((_CH + 1, _CHUNK), jnp.int32),
            pltpu.VMEM((_CH, _CHUNK), jnp.int32),
            pltpu.VMEM((_CHUNK, F), dtype),
            pltpu.VMEM((_CHUNK, F), dtype),
            pltpu.SemaphoreType.DMA,
            pltpu.SemaphoreType.DMA,
            pltpu.VMEM_SHARED((_NPAD, F), dtype),
        ],
    )(body)


_agg64 = _make_agg(64, jnp.bfloat16)

_BLK = 2000
_G = _N // _BLK


def _mm1_body(x_ref, w_ref, d0_ref, d1_ref, o_ref):
    dis = lax.rsqrt(d0_ref[...] + d1_ref[...] + 1.0)
    y = dis * jnp.dot(x_ref[...], w_ref[...],
                      preferred_element_type=jnp.float32)
    o_ref[...] = y.astype(jnp.bfloat16)


_mm1 = pl.pallas_call(
    _mm1_body,
    grid=(_G,),
    in_specs=[
        pl.BlockSpec((_BLK, 128), lambda i: (i, 0)),
        pl.BlockSpec((128, 64), lambda i: (0, 0)),
        pl.BlockSpec((_BLK, 1), lambda i: (i, 0)),
        pl.BlockSpec((_BLK, 1), lambda i: (i, 0)),
    ],
    out_specs=pl.BlockSpec((_BLK, 64), lambda i: (i, 0)),
    out_shape=jax.ShapeDtypeStruct((_N, 64), jnp.bfloat16),
)


def _mm2_body(p0_ref, p1_ref, y1_ref, d0_ref, d1_ref, b1_ref, w2_ref,
              oa_ref, ob_ref):
    dis = lax.rsqrt(d0_ref[...] + d1_ref[...] + 1.0)
    agg = (p0_ref[...].astype(jnp.float32) + p1_ref[...].astype(jnp.float32)
           + y1_ref[...].astype(jnp.float32))
    h = jnp.maximum(dis * agg + b1_ref[...], 0.0)
    y2 = dis * jnp.dot(h, w2_ref[...], preferred_element_type=jnp.float32)
    oa_ref[...] = y2[:, :64].astype(jnp.bfloat16)
    ob_ref[...] = y2[:, 64:].astype(jnp.bfloat16)


_mm2 = pl.pallas_call(
    _mm2_body,
    grid=(_G,),
    in_specs=[
        pl.BlockSpec((_BLK, 64), lambda i: (i, 0)),
        pl.BlockSpec((_BLK, 64), lambda i: (i, 0)),
        pl.BlockSpec((_BLK, 64), lambda i: (i, 0)),
        pl.BlockSpec((_BLK, 1), lambda i: (i, 0)),
        pl.BlockSpec((_BLK, 1), lambda i: (i, 0)),
        pl.BlockSpec((1, 64), lambda i: (0, 0)),
        pl.BlockSpec((64, 128), lambda i: (0, 0)),
    ],
    out_specs=[
        pl.BlockSpec((_BLK, 64), lambda i: (i, 0)),
        pl.BlockSpec((_BLK, 64), lambda i: (i, 0)),
    ],
    out_shape=[
        jax.ShapeDtypeStruct((_N, 64), jnp.bfloat16),
        jax.ShapeDtypeStruct((_N, 64), jnp.bfloat16),
    ],
)


def _fin_body(qa0_ref, qa1_ref, qb0_ref, qb1_ref, ya_ref, yb_ref,
              d0_ref, d1_ref, b2_ref, o_ref):
    dis = lax.rsqrt(d0_ref[...] + d1_ref[...] + 1.0)
    f32 = jnp.float32
    oa = dis * (qa0_ref[...].astype(f32) + qa1_ref[...].astype(f32)
                + ya_ref[...].astype(f32))
    ob = dis * (qb0_ref[...].astype(f32) + qb1_ref[...].astype(f32)
                + yb_ref[...].astype(f32))
    o_ref[...] = jnp.concatenate([oa, ob], axis=1) + b2_ref[...]


_fin = pl.pallas_call(
    _fin_body,
    grid=(_G,),
    in_specs=[
        pl.BlockSpec((_BLK, 64), lambda i: (i, 0)),
        pl.BlockSpec((_BLK, 64), lambda i: (i, 0)),
        pl.BlockSpec((_BLK, 64), lambda i: (i, 0)),
        pl.BlockSpec((_BLK, 64), lambda i: (i, 0)),
        pl.BlockSpec((_BLK, 64), lambda i: (i, 0)),
        pl.BlockSpec((_BLK, 64), lambda i: (i, 0)),
        pl.BlockSpec((_BLK, 1), lambda i: (i, 0)),
        pl.BlockSpec((_BLK, 1), lambda i: (i, 0)),
        pl.BlockSpec((1, 128), lambda i: (0, 0)),
    ],
    out_specs=pl.BlockSpec((_BLK, 128), lambda i: (i, 0)),
    out_shape=jax.ShapeDtypeStruct((_N, 128), jnp.float32),
)


def kernel(x, edge_index, W1, b1, W2, b2):
    row = edge_index[0].astype(jnp.int32)
    col = edge_index[1].astype(jnp.int32)
    pad = _EPAD - _E
    rowp = jnp.concatenate([row, jnp.zeros((pad,), jnp.int32)])
    colp = jnp.concatenate([col, jnp.full((pad,), _N, jnp.int32)])
    rowp = rowp.reshape(_NW, _CH, _CHUNK)
    colp = colp.reshape(_NW, _CH, _CHUNK)
    rows_g = jnp.concatenate(
        [rowp, jnp.zeros((_NW, 1, _CHUNK), jnp.int32)], axis=1)
    zeros1 = jnp.zeros((_NPAD,), jnp.float32)
    zeros64 = jnp.zeros((_NPAD, 64), jnp.bfloat16)

    deg = _deg_kernel(colp, zeros1)
    d0 = deg[0, :_N, None]
    d1 = deg[1, :_N, None]

    y1 = _mm1(x, W1, d0, d1)
    p = _agg64(y1, rows_g, colp, zeros64)
    ya, yb = _mm2(p[0, :_N], p[1, :_N], y1, d0, d1, b1[None, :], W2)
    qa = _agg64(ya, rows_g, colp, zeros64)
    qb = _agg64(yb, rows_g, colp, zeros64)
    out = _fin(qa[0, :_N], qa[1, :_N], qb[0, :_N], qb[1, :_N],
               ya, yb, d0, d1, b2[None, :])
    return out


# speedup vs baseline: 15.3504x; 1.0317x over previous
import functools

import jax
import jax.numpy as jnp
from jax import lax
from jax.experimental import pallas as pl
from jax.experimental.pallas import tpu as pltpu
from jax.experimental.pallas import tpu_sc as plsc

_N = 10000
_E = 320000
_NW = 32
_RPT = 640
_NPAD = 10240
_CHUNK = 128
_CH = 80
_EPAD = _NW * _CH * _CHUNK

_mesh = plsc.VectorSubcoreMesh(core_axis_name="c", subcore_axis_name="s")
_sc_params = pltpu.CompilerParams(use_tc_tiling_on_sc=False)


def _deg_body(cols_hbm, zeros_hbm, out_hbm, colv, onesv, acc):
    cid = lax.axis_index("c")
    sid = lax.axis_index("s")
    wid = sid * 2 + cid
    sl = pl.ds(sid * _RPT, _RPT)
    pltpu.sync_copy(zeros_hbm.at[sl], acc.at[sl])
    pltpu.sync_copy(cols_hbm.at[wid], colv)
    for k in range(_CHUNK // 16):
        onesv[pl.ds(k * 16, 16)] = jnp.full((16,), 1.0, jnp.float32)
    plsc.subcore_barrier()

    def body(j, carry):
        pltpu.sync_copy(onesv, acc.at[colv.at[j]], add=True)
        return carry

    lax.fori_loop(0, _CH, body, 0)
    plsc.subcore_barrier()
    pltpu.sync_copy(acc.at[sl], out_hbm.at[cid, sl])


_deg_kernel = functools.partial(
    pl.kernel,
    out_type=jax.ShapeDtypeStruct((2, _NPAD), jnp.float32),
    mesh=_mesh,
    compiler_params=_sc_params,
    scratch_types=[
        pltpu.VMEM((_CH, _CHUNK), jnp.int32),
        pltpu.VMEM((_CHUNK,), jnp.float32),
        pltpu.VMEM_SHARED((_NPAD,), jnp.float32),
    ],
)(_deg_body)


def _make_agg(F, dtype):
    def body(y_hbm, rows_hbm, cols_hbm, zeros_hbm, out_hbm,
             rowv, colv, bufa, bufb, sema, semb, acc):
        cid = lax.axis_index("c")
        sid = lax.axis_index("s")
        wid = sid * 2 + cid
        sl = pl.ds(sid * _RPT, _RPT)
        pltpu.sync_copy(zeros_hbm.at[sl], acc.at[sl])
        pltpu.sync_copy(rows_hbm.at[wid], rowv)
        pltpu.sync_copy(cols_hbm.at[wid], colv)
        plsc.subcore_barrier()

        pltpu.async_copy(y_hbm.at[rowv.at[0]], bufa, sema)

        def body2(i, carry):
            j = 2 * i
            pltpu.make_async_copy(y_hbm.at[rowv.at[j]], bufa, sema).wait()
            pltpu.async_copy(y_hbm.at[rowv.at[j + 1]], bufb, semb)
            pltpu.sync_copy(bufa, acc.at[colv.at[j]], add=True)
            pltpu.make_async_copy(y_hbm.at[rowv.at[j + 1]], bufb, semb).wait()
            pltpu.async_copy(y_hbm.at[rowv.at[j + 2]], bufa, sema)
            pltpu.sync_copy(bufb, acc.at[colv.at[j + 1]], add=True)
            return carry

        lax.fori_loop(0, _CH // 2, body2, 0)
        pltpu.make_async_copy(y_hbm.at[rowv.at[_CH]], bufa, sema).wait()
        plsc.subcore_barrier()
        pltpu.sync_copy(acc.at[sl], out_hbm.at[cid, sl])

    return functools.partial(
        pl.kernel,
        out_type=jax.ShapeDtypeStruct((2, _NPAD, F), dtype),
        mesh=_mesh,
        compiler_params=_sc_params,
        scratch_types=[
            pltpu.VMEM((_CH + 1, _CHUNK), jnp.int32),
            pltpu.VMEM((_CH, _CHUNK), jnp.int32),
            pltpu.VMEM((_CHUNK, F), dtype),
            pltpu.VMEM((_CHUNK, F), dtype),
            pltpu.SemaphoreType.DMA,
            pltpu.SemaphoreType.DMA,
            pltpu.VMEM_SHARED((_NPAD, F), dtype),
        ],
    )(body)


_agg64 = _make_agg(64, jnp.bfloat16)
_agg128 = _make_agg(128, jnp.bfloat16)

_BLK = 2000
_G = _N // _BLK


def _mm1_body(x_ref, w_ref, d0_ref, d1_ref, o_ref):
    dis = lax.rsqrt(d0_ref[...] + d1_ref[...] + 1.0)
    y = dis * jnp.dot(x_ref[...], w_ref[...],
                      preferred_element_type=jnp.float32)
    o_ref[...] = y.astype(jnp.bfloat16)


_mm1 = pl.pallas_call(
    _mm1_body,
    grid=(_G,),
    in_specs=[
        pl.BlockSpec((_BLK, 128), lambda i: (i, 0)),
        pl.BlockSpec((128, 64), lambda i: (0, 0)),
        pl.BlockSpec((_BLK, 1), lambda i: (i, 0)),
        pl.BlockSpec((_BLK, 1), lambda i: (i, 0)),
    ],
    out_specs=pl.BlockSpec((_BLK, 64), lambda i: (i, 0)),
    out_shape=jax.ShapeDtypeStruct((_N, 64), jnp.bfloat16),
)


def _mm2_body(p0_ref, p1_ref, y1_ref, d0_ref, d1_ref, b1_ref, w2_ref,
              oa_ref):
    dis = lax.rsqrt(d0_ref[...] + d1_ref[...] + 1.0)
    agg = (p0_ref[...].astype(jnp.float32) + p1_ref[...].astype(jnp.float32)
           + y1_ref[...].astype(jnp.float32))
    h = jnp.maximum(dis * agg + b1_ref[...], 0.0)
    y2 = dis * jnp.dot(h, w2_ref[...], preferred_element_type=jnp.float32)
    oa_ref[...] = y2.astype(jnp.bfloat16)


_mm2 = pl.pallas_call(
    _mm2_body,
    grid=(_G,),
    in_specs=[
        pl.BlockSpec((_BLK, 64), lambda i: (i, 0)),
        pl.BlockSpec((_BLK, 64), lambda i: (i, 0)),
        pl.BlockSpec((_BLK, 64), lambda i: (i, 0)),
        pl.BlockSpec((_BLK, 1), lambda i: (i, 0)),
        pl.BlockSpec((_BLK, 1), lambda i: (i, 0)),
        pl.BlockSpec((1, 64), lambda i: (0, 0)),
        pl.BlockSpec((64, 128), lambda i: (0, 0)),
    ],
    out_specs=pl.BlockSpec((_BLK, 128), lambda i: (i, 0)),
    out_shape=jax.ShapeDtypeStruct((_N, 128), jnp.bfloat16),
)


def _fin_body(q0_ref, q1_ref, y2_ref, d0_ref, d1_ref, b2_ref, o_ref):
    dis = lax.rsqrt(d0_ref[...] + d1_ref[...] + 1.0)
    f32 = jnp.float32
    o_ref[...] = dis * (q0_ref[...].astype(f32) + q1_ref[...].astype(f32)
                        + y2_ref[...].astype(f32)) + b2_ref[...]


_fin = pl.pallas_call(
    _fin_body,
    grid=(_G,),
    in_specs=[
        pl.BlockSpec((_BLK, 128), lambda i: (i, 0)),
        pl.BlockSpec((_BLK, 128), lambda i: (i, 0)),
        pl.BlockSpec((_BLK, 128), lambda i: (i, 0)),
        pl.BlockSpec((_BLK, 1), lambda i: (i, 0)),
        pl.BlockSpec((_BLK, 1), lambda i: (i, 0)),
        pl.BlockSpec((1, 128), lambda i: (0, 0)),
    ],
    out_specs=pl.BlockSpec((_BLK, 128), lambda i: (i, 0)),
    out_shape=jax.ShapeDtypeStruct((_N, 128), jnp.float32),
)


def kernel(x, edge_index, W1, b1, W2, b2):
    row = edge_index[0].astype(jnp.int32)
    col = edge_index[1].astype(jnp.int32)
    pad = _EPAD - _E
    rowp = jnp.concatenate([row, jnp.zeros((pad,), jnp.int32)])
    colp = jnp.concatenate([col, jnp.full((pad,), _N, jnp.int32)])
    rowp = rowp.reshape(_NW, _CH, _CHUNK)
    colp = colp.reshape(_NW, _CH, _CHUNK)
    rows_g = jnp.concatenate(
        [rowp, jnp.zeros((_NW, 1, _CHUNK), jnp.int32)], axis=1)
    zeros1 = jnp.zeros((_NPAD,), jnp.float32)
    zeros64 = jnp.zeros((_NPAD, 64), jnp.bfloat16)
    zeros128 = jnp.zeros((_NPAD, 128), jnp.bfloat16)

    deg = _deg_kernel(colp, zeros1)
    d0 = deg[0, :_N, None]
    d1 = deg[1, :_N, None]

    y1 = _mm1(x, W1, d0, d1)
    p = _agg64(y1, rows_g, colp, zeros64)
    y2 = _mm2(p[0, :_N], p[1, :_N], y1, d0, d1, b1[None, :], W2)
    q = _agg128(y2, rows_g, colp, zeros128)
    out = _fin(q[0, :_N], q[1, :_N], y2, d0, d1, b2[None, :])
    return out


# speedup vs baseline: 21.4960x; 1.4004x over previous
import functools

import jax
import jax.numpy as jnp
from jax import lax
from jax.experimental import pallas as pl
from jax.experimental.pallas import tpu as pltpu
from jax.experimental.pallas import tpu_sc as plsc

_N = 10000
_E = 320000
_NW = 32
_RPT = 640
_NPAD = 10240
_CHUNK = 128
_CH = 80
_EPAD = _NW * _CH * _CHUNK

_mesh = plsc.VectorSubcoreMesh(core_axis_name="c", subcore_axis_name="s")
_sc_params = pltpu.CompilerParams(use_tc_tiling_on_sc=False)


def _deg_body(cols_hbm, zeros_hbm, out_hbm, colv, onesv, acc):
    cid = lax.axis_index("c")
    sid = lax.axis_index("s")
    wid = sid * 2 + cid
    sl = pl.ds(sid * _RPT, _RPT)
    pltpu.sync_copy(zeros_hbm.at[sl], acc.at[sl])
    pltpu.sync_copy(cols_hbm.at[wid], colv)
    for k in range(_CHUNK // 16):
        onesv[pl.ds(k * 16, 16)] = jnp.full((16,), 1.0, jnp.float32)
    plsc.subcore_barrier()

    def body(j, carry):
        pltpu.sync_copy(onesv, acc.at[colv.at[j]], add=True)
        return carry

    lax.fori_loop(0, _CH, body, 0)
    plsc.subcore_barrier()
    pltpu.sync_copy(acc.at[sl], out_hbm.at[cid, sl])


_deg_kernel = functools.partial(
    pl.kernel,
    out_type=jax.ShapeDtypeStruct((2, _NPAD), jnp.float32),
    mesh=_mesh,
    compiler_params=_sc_params,
    scratch_types=[
        pltpu.VMEM((_CH, _CHUNK), jnp.int32),
        pltpu.VMEM((_CHUNK,), jnp.float32),
        pltpu.VMEM_SHARED((_NPAD,), jnp.float32),
    ],
)(_deg_body)


def _make_agg(F, dtype):
    def body(y_hbm, rows_hbm, cols_hbm, zeros_hbm, out_hbm,
             rowv, colv, bufa, bufb, sema, semb, acc):
        cid = lax.axis_index("c")
        sid = lax.axis_index("s")
        wid = sid * 2 + cid
        sl = pl.ds(sid * _RPT, _RPT)
        pltpu.sync_copy(zeros_hbm.at[sl], acc.at[sl])
        pltpu.sync_copy(rows_hbm.at[wid], rowv)
        pltpu.sync_copy(cols_hbm.at[wid], colv)
        plsc.subcore_barrier()

        pltpu.async_copy(y_hbm.at[rowv.at[0]], bufa, sema)

        def body2(i, carry):
            j = 2 * i
            pltpu.make_async_copy(y_hbm.at[rowv.at[j]], bufa, sema).wait()
            pltpu.async_copy(y_hbm.at[rowv.at[j + 1]], bufb, semb)
            pltpu.sync_copy(bufa, acc.at[colv.at[j]], add=True)
            pltpu.make_async_copy(y_hbm.at[rowv.at[j + 1]], bufb, semb).wait()
            pltpu.async_copy(y_hbm.at[rowv.at[j + 2]], bufa, sema)
            pltpu.sync_copy(bufb, acc.at[colv.at[j + 1]], add=True)
            return carry

        lax.fori_loop(0, _CH // 2, body2, 0)
        pltpu.make_async_copy(y_hbm.at[rowv.at[_CH]], bufa, sema).wait()
        plsc.subcore_barrier()
        pltpu.sync_copy(acc.at[sl], out_hbm.at[cid, sl])

    return functools.partial(
        pl.kernel,
        out_type=jax.ShapeDtypeStruct((2, _NPAD, F), dtype),
        mesh=_mesh,
        compiler_params=_sc_params,
        scratch_types=[
            pltpu.VMEM((_CH + 1, _CHUNK), jnp.int32),
            pltpu.VMEM((_CH, _CHUNK), jnp.int32),
            pltpu.VMEM((_CHUNK, F), dtype),
            pltpu.VMEM((_CHUNK, F), dtype),
            pltpu.SemaphoreType.DMA,
            pltpu.SemaphoreType.DMA,
            pltpu.VMEM_SHARED((_NPAD, F), dtype),
        ],
    )(body)


_agg64 = _make_agg(64, jnp.bfloat16)

_BLK = 2000
_G = _N // _BLK


def _mm1_body(x_ref, w_ref, d0_ref, d1_ref, o_ref):
    dis = lax.rsqrt(d0_ref[...] + d1_ref[...] + 1.0)
    y = dis * jnp.dot(x_ref[...], w_ref[...],
                      preferred_element_type=jnp.float32)
    o_ref[...] = y.astype(jnp.bfloat16)


_mm1 = pl.pallas_call(
    _mm1_body,
    grid=(_G,),
    in_specs=[
        pl.BlockSpec((_BLK, 128), lambda i: (i, 0)),
        pl.BlockSpec((128, 64), lambda i: (0, 0)),
        pl.BlockSpec((_BLK, 1), lambda i: (i, 0)),
        pl.BlockSpec((_BLK, 1), lambda i: (i, 0)),
    ],
    out_specs=pl.BlockSpec((_BLK, 64), lambda i: (i, 0)),
    out_shape=jax.ShapeDtypeStruct((_N, 64), jnp.bfloat16),
)


def _mid_body(p0_ref, p1_ref, y1_ref, d0_ref, d1_ref, b1_ref, o_ref):
    dis = lax.rsqrt(d0_ref[...] + d1_ref[...] + 1.0)
    agg = (p0_ref[...].astype(jnp.float32) + p1_ref[...].astype(jnp.float32)
           + y1_ref[...].astype(jnp.float32))
    h = jnp.maximum(dis * agg + b1_ref[...], 0.0)
    o_ref[...] = (dis * h).astype(jnp.bfloat16)


_mid = pl.pallas_call(
    _mid_body,
    grid=(_G,),
    in_specs=[
        pl.BlockSpec((_BLK, 64), lambda i: (i, 0)),
        pl.BlockSpec((_BLK, 64), lambda i: (i, 0)),
        pl.BlockSpec((_BLK, 64), lambda i: (i, 0)),
        pl.BlockSpec((_BLK, 1), lambda i: (i, 0)),
        pl.BlockSpec((_BLK, 1), lambda i: (i, 0)),
        pl.BlockSpec((1, 64), lambda i: (0, 0)),
    ],
    out_specs=pl.BlockSpec((_BLK, 64), lambda i: (i, 0)),
    out_shape=jax.ShapeDtypeStruct((_N, 64), jnp.bfloat16),
)


def _fin_body(q0_ref, q1_ref, z_ref, d0_ref, d1_ref, b2_ref, w2_ref, o_ref):
    dis = lax.rsqrt(d0_ref[...] + d1_ref[...] + 1.0)
    f32 = jnp.float32
    agg = (q0_ref[...].astype(f32) + q1_ref[...].astype(f32)
           + z_ref[...].astype(f32))
    o_ref[...] = dis * jnp.dot(agg, w2_ref[...],
                               preferred_element_type=f32) + b2_ref[...]


_fin = pl.pallas_call(
    _fin_body,
    grid=(_G,),
    in_specs=[
        pl.BlockSpec((_BLK, 64), lambda i: (i, 0)),
        pl.BlockSpec((_BLK, 64), lambda i: (i, 0)),
        pl.BlockSpec((_BLK, 64), lambda i: (i, 0)),
        pl.BlockSpec((_BLK, 1), lambda i: (i, 0)),
        pl.BlockSpec((_BLK, 1), lambda i: (i, 0)),
        pl.BlockSpec((1, 128), lambda i: (0, 0)),
        pl.BlockSpec((64, 128), lambda i: (0, 0)),
    ],
    out_specs=pl.BlockSpec((_BLK, 128), lambda i: (i, 0)),
    out_shape=jax.ShapeDtypeStruct((_N, 128), jnp.float32),
)


def kernel(x, edge_index, W1, b1, W2, b2):
    row = edge_index[0].astype(jnp.int32)
    col = edge_index[1].astype(jnp.int32)
    pad = _EPAD - _E
    rowp = jnp.concatenate([row, jnp.zeros((pad,), jnp.int32)])
    colp = jnp.concatenate([col, jnp.full((pad,), _N, jnp.int32)])
    rowp = rowp.reshape(_NW, _CH, _CHUNK)
    colp = colp.reshape(_NW, _CH, _CHUNK)
    rows_g = jnp.concatenate(
        [rowp, jnp.zeros((_NW, 1, _CHUNK), jnp.int32)], axis=1)
    zeros1 = jnp.zeros((_NPAD,), jnp.float32)
    zeros64 = jnp.zeros((_NPAD, 64), jnp.bfloat16)

    deg = _deg_kernel(colp, zeros1)
    d0 = deg[0, :_N, None]
    d1 = deg[1, :_N, None]

    y1 = _mm1(x, W1, d0, d1)
    p = _agg64(y1, rows_g, colp, zeros64)
    z = _mid(p[0, :_N], p[1, :_N], y1, d0, d1, b1[None, :])
    q = _agg64(z, rows_g, colp, zeros64)
    out = _fin(q[0, :_N], q[1, :_N], z, d0, d1, b2[None, :], W2)
    return out


# speedup vs baseline: 21.8544x; 1.0167x over previous
import functools

import jax
import jax.numpy as jnp
from jax import lax
from jax.experimental import pallas as pl
from jax.experimental.pallas import tpu as pltpu
from jax.experimental.pallas import tpu_sc as plsc

_N = 10000
_E = 320000
_RPT = 640
_NPAD = 10240
_CHUNK = 128
_CHD = 80
_CHF = 160
_EPAD = 32 * _CHD * _CHUNK

_mesh = plsc.VectorSubcoreMesh(core_axis_name="c", subcore_axis_name="s")
_sc_params = pltpu.CompilerParams(use_tc_tiling_on_sc=False)


def _deg_body(cols_hbm, zeros_hbm, out_hbm, colv, onesv, acc):
    cid = lax.axis_index("c")
    sid = lax.axis_index("s")
    wid = sid * 2 + cid
    sl = pl.ds(sid * _RPT, _RPT)
    pltpu.sync_copy(zeros_hbm.at[sl], acc.at[sl])
    pltpu.sync_copy(cols_hbm.at[wid], colv)
    for k in range(_CHUNK // 16):
        onesv[pl.ds(k * 16, 16)] = jnp.full((16,), 1.0, jnp.float32)
    plsc.subcore_barrier()

    def body(j, carry):
        pltpu.sync_copy(onesv, acc.at[colv.at[j]], add=True)
        return carry

    lax.fori_loop(0, _CHD, body, 0)
    plsc.subcore_barrier()
    pltpu.sync_copy(acc.at[sl], out_hbm.at[cid, sl])


_deg_kernel = functools.partial(
    pl.kernel,
    out_type=jax.ShapeDtypeStruct((2, _NPAD), jnp.float32),
    mesh=_mesh,
    compiler_params=_sc_params,
    scratch_types=[
        pltpu.VMEM((_CHD, _CHUNK), jnp.int32),
        pltpu.VMEM((_CHUNK,), jnp.float32),
        pltpu.VMEM_SHARED((_NPAD,), jnp.float32),
    ],
)(_deg_body)


def _edge_loop(table_hbm, rowv, colv, bufa, bufb, sema, semb, acc):
    pltpu.async_copy(table_hbm.at[rowv.at[0]], bufa, sema)

    def body(i, carry):
        j = 2 * i
        pltpu.make_async_copy(table_hbm.at[rowv.at[j]], bufa, sema).wait()
        pltpu.async_copy(table_hbm.at[rowv.at[j + 1]], bufb, semb)
        pltpu.sync_copy(bufa, acc.at[colv.at[j]], add=True)
        pltpu.make_async_copy(table_hbm.at[rowv.at[j + 1]], bufb, semb).wait()
        pltpu.async_copy(table_hbm.at[rowv.at[j + 2]], bufa, sema)
        pltpu.sync_copy(bufb, acc.at[colv.at[j + 1]], add=True)
        return carry

    lax.fori_loop(0, _CHF // 2, body, 0)
    pltpu.make_async_copy(table_hbm.at[rowv.at[_CHF]], bufa, sema).wait()


def _fused_body(y_hbm, rows_hbm, cols_hbm, zeros_hbm, dis_hbm, b1_hbm,
                z_hbm, out_hbm,
                rowv, colv, bufa, bufb, sema, semb,
                accv, y1v, zv, b1v, diss, acc1, acc2):
    cid = lax.axis_index("c")
    sid = lax.axis_index("s")
    sl = pl.ds(sid * _RPT, _RPT)
    pltpu.sync_copy(zeros_hbm.at[sl], acc1.at[sl])
    pltpu.sync_copy(zeros_hbm.at[sl], acc2.at[sl])
    pltpu.sync_copy(rows_hbm.at[sid], rowv)
    pltpu.sync_copy(cols_hbm.at[sid], colv)
    pltpu.sync_copy(dis_hbm.at[sl], diss)
    pltpu.sync_copy(b1_hbm.at[cid], b1v)
    plsc.subcore_barrier()

    _edge_loop(y_hbm.at[cid], rowv, colv, bufa, bufb, sema, semb, acc1)
    plsc.subcore_barrier()

    pltpu.sync_copy(acc1.at[sl], accv)
    pltpu.sync_copy(y_hbm.at[cid, sl], y1v)
    b1r = b1v[...]
    zero = jnp.zeros((32,), jnp.bfloat16)

    def mid(r, carry):
        d = diss[r]
        zv[r] = d * jnp.maximum(d * (accv[r] + y1v[r]) + b1r, zero)
        return carry

    lax.fori_loop(0, _RPT, mid, 0)
    pltpu.sync_copy(zv, z_hbm.at[cid, sl])
    plsc.subcore_barrier()

    _edge_loop(z_hbm.at[cid], rowv, colv, bufa, bufb, sema, semb, acc2)
    plsc.subcore_barrier()
    pltpu.sync_copy(acc2.at[sl], out_hbm.at[cid, sl])


_fused_kernel = functools.partial(
    pl.kernel,
    out_type=[
        jax.ShapeDtypeStruct((2, _NPAD, 32), jnp.bfloat16),
        jax.ShapeDtypeStruct((2, _NPAD, 32), jnp.bfloat16),
    ],
    mesh=_mesh,
    compiler_params=_sc_params,
    scratch_types=[
        pltpu.VMEM((_CHF + 1, _CHUNK), jnp.int32),
        pltpu.VMEM((_CHF, _CHUNK), jnp.int32),
        pltpu.VMEM((_CHUNK, 32), jnp.bfloat16),
        pltpu.VMEM((_CHUNK, 32), jnp.bfloat16),
        pltpu.SemaphoreType.DMA,
        pltpu.SemaphoreType.DMA,
        pltpu.VMEM((_RPT, 32), jnp.bfloat16),
        pltpu.VMEM((_RPT, 32), jnp.bfloat16),
        pltpu.VMEM((_RPT, 32), jnp.bfloat16),
        pltpu.VMEM((32,), jnp.bfloat16),
        pltpu.VMEM((_RPT, 32), jnp.bfloat16),
        pltpu.VMEM_SHARED((_NPAD, 32), jnp.bfloat16),
        pltpu.VMEM_SHARED((_NPAD, 32), jnp.bfloat16),
    ],
)(_fused_body)

_BLKM = 2048
_GM = _NPAD // _BLKM


def _mm1_body(x_ref, w_ref, d0_ref, d1_ref, ya_ref, yb_ref, dis_ref):
    dis = lax.rsqrt(d0_ref[...] + d1_ref[...] + 1.0)
    y = dis * jnp.dot(x_ref[...], w_ref[...],
                      preferred_element_type=jnp.float32)
    ya_ref[...] = y[:, :32].astype(jnp.bfloat16)
    yb_ref[...] = y[:, 32:].astype(jnp.bfloat16)
    dis_ref[...] = jnp.broadcast_to(dis.astype(jnp.bfloat16),
                                    (dis.shape[0], 32))


_mm1 = pl.pallas_call(
    _mm1_body,
    grid=(_GM,),
    in_specs=[
        pl.BlockSpec((_BLKM, 128), lambda i: (i, 0)),
        pl.BlockSpec((128, 64), lambda i: (0, 0)),
        pl.BlockSpec((_BLKM, 1), lambda i: (i, 0)),
        pl.BlockSpec((_BLKM, 1), lambda i: (i, 0)),
    ],
    out_specs=[
        pl.BlockSpec((_BLKM, 32), lambda i: (i, 0)),
        pl.BlockSpec((_BLKM, 32), lambda i: (i, 0)),
        pl.BlockSpec((_BLKM, 32), lambda i: (i, 0)),
    ],
    out_shape=[
        jax.ShapeDtypeStruct((_NPAD, 32), jnp.bfloat16),
        jax.ShapeDtypeStruct((_NPAD, 32), jnp.bfloat16),
        jax.ShapeDtypeStruct((_NPAD, 32), jnp.bfloat16),
    ],
)

_BLK = 2000
_G = _N // _BLK


def _fin_body(a0_ref, a1_ref, z0_ref, z1_ref, d0_ref, d1_ref, b2_ref,
              w2_ref, o_ref):
    dis = lax.rsqrt(d0_ref[...] + d1_ref[...] + 1.0)
    f32 = jnp.float32
    agg = jnp.concatenate(
        [a0_ref[...].astype(f32) + z0_ref[...].astype(f32),
         a1_ref[...].astype(f32) + z1_ref[...].astype(f32)], axis=1)
    o_ref[...] = dis * jnp.dot(agg, w2_ref[...],
                               preferred_element_type=f32) + b2_ref[...]


_fin = pl.pallas_call(
    _fin_body,
    grid=(_G,),
    in_specs=[
        pl.BlockSpec((_BLK, 32), lambda i: (i, 0)),
        pl.BlockSpec((_BLK, 32), lambda i: (i, 0)),
        pl.BlockSpec((_BLK, 32), lambda i: (i, 0)),
        pl.BlockSpec((_BLK, 32), lambda i: (i, 0)),
        pl.BlockSpec((_BLK, 1), lambda i: (i, 0)),
        pl.BlockSpec((_BLK, 1), lambda i: (i, 0)),
        pl.BlockSpec((1, 128), lambda i: (0, 0)),
        pl.BlockSpec((64, 128), lambda i: (0, 0)),
    ],
    out_specs=pl.BlockSpec((_BLK, 128), lambda i: (i, 0)),
    out_shape=jax.ShapeDtypeStruct((_N, 128), jnp.float32),
)


def kernel(x, edge_index, W1, b1, W2, b2):
    row = edge_index[0].astype(jnp.int32)
    col = edge_index[1].astype(jnp.int32)
    pad = _EPAD - _E
    rowp = jnp.concatenate([row, jnp.zeros((pad,), jnp.int32)])
    colp = jnp.concatenate([col, jnp.full((pad,), _N, jnp.int32)])
    colp32 = colp.reshape(32, _CHD, _CHUNK)
    rowp16 = rowp.reshape(16, _CHF, _CHUNK)
    colp16 = colp.reshape(16, _CHF, _CHUNK)
    rows_g = jnp.concatenate(
        [rowp16, jnp.zeros((16, 1, _CHUNK), jnp.int32)], axis=1)
    zeros1 = jnp.zeros((_NPAD,), jnp.float32)
    zeros32 = jnp.zeros((_NPAD, 32), jnp.bfloat16)
    x_pad = jnp.pad(x, ((0, _NPAD - _N), (0, 0)))
    b1h = b1.astype(jnp.bfloat16).reshape(2, 32)

    deg = _deg_kernel(colp32, zeros1)
    d0 = deg[0][:, None]
    d1 = deg[1][:, None]

    ya, yb, dis16 = _mm1(x_pad, W1, d0, d1)
    ystk = jnp.stack([ya, yb])
    z, a2 = _fused_kernel(ystk, rows_g, colp16, zeros32, dis16, b1h)
    out = _fin(a2[0, :_N], a2[1, :_N], z[0, :_N], z[1, :_N],
               d0[:_N], d1[:_N], b2[None, :], W2)
    return out
